# baseline (speedup 1.0000x reference)
import jax
import jax.numpy as jnp
from jax import lax
from jax.experimental import pallas as pl
from jax.experimental.pallas import tpu as pltpu

ALPHA = 1.0
GAMMA = 2.0
N_ROWS = 16384
N_CLS = 1000
BLK = 4096
CH = 8


def _focal_body(x_ref, t_ref, out_ref):
    i = pl.program_id(0)
    t = t_ref[...]
    sub_iota = lax.broadcasted_iota(jnp.int32, (CH, BLK), 0)
    m_acc = jnp.full((CH, BLK), -jnp.inf, jnp.float32)
    g_acc = jnp.zeros((CH, BLK), jnp.float32)
    for c in range(0, N_CLS, CH):
        xc = x_ref[c:c + CH, :]
        m_acc = jnp.maximum(m_acc, xc)
        g_acc = g_acc + jnp.where(sub_iota + c == t, xc, 0.0)
    m = jnp.max(m_acc, axis=0, keepdims=True)
    xt = jnp.sum(g_acc, axis=0, keepdims=True)

    s_acc = jnp.zeros((CH, BLK), jnp.float32)
    for c in range(0, N_CLS, CH):
        xc = x_ref[c:c + CH, :]
        s_acc = s_acc + jnp.exp(xc - m)
    s = jnp.sum(s_acc, axis=0, keepdims=True)

    logpt = (xt - m) - jnp.log(s)
    pt = jnp.exp(logpt)
    loss = -ALPHA * (1.0 - pt) * (1.0 - pt) * logpt
    bsum = jnp.sum(loss) * (1.0 / N_ROWS)

    @pl.when(i == 0)
    def _():
        out_ref[0, 0] = 0.0

    out_ref[0, 0] += bsum


def kernel(inputs, targets):
    xt_view = inputs.T
    t2d = targets.astype(jnp.int32).reshape(1, N_ROWS)
    out = pl.pallas_call(
        _focal_body,
        grid=(N_ROWS // BLK,),
        in_specs=[
            pl.BlockSpec((N_CLS, BLK), lambda i: (0, i)),
            pl.BlockSpec((1, BLK), lambda i: (0, i)),
        ],
        out_specs=pl.BlockSpec(
            (1, 1), lambda i: (0, 0), memory_space=pltpu.SMEM
        ),
        out_shape=jax.ShapeDtypeStruct((1, 1), jnp.float32),
        compiler_params=pltpu.CompilerParams(
            dimension_semantics=("arbitrary",),
        ),
    )(xt_view, t2d)
    return out[0, 0]


# speedup vs baseline: 1.2867x; 1.2867x over previous
import jax
import jax.numpy as jnp
from jax import lax
from jax.experimental import pallas as pl
from jax.experimental.pallas import tpu as pltpu

ALPHA = 1.0
GAMMA = 2.0
N_ROWS = 16384
N_CLS = 1000
BLK = 2048
CH = 8


def _focal_body(x_ref, t_ref, out_ref):
    i = pl.program_id(0)
    t = t_ref[...]
    sub_iota = lax.broadcasted_iota(jnp.int32, (CH, BLK), 0)
    m_acc = jnp.full((CH, BLK), -jnp.inf, jnp.float32)
    for c in range(0, N_CLS, CH):
        m_acc = jnp.maximum(m_acc, x_ref[c:c + CH, :])
    m = jnp.max(m_acc, axis=0, keepdims=True)

    g_acc = jnp.zeros((CH, BLK), jnp.float32)
    s_acc = jnp.zeros((CH, BLK), jnp.float32)
    for c in range(0, N_CLS, CH):
        xc = x_ref[c:c + CH, :]
        g_acc = g_acc + jnp.where(sub_iota + c == t, xc, 0.0)
        s_acc = s_acc + jnp.exp(xc - m)
    xt = jnp.sum(g_acc, axis=0, keepdims=True)
    s = jnp.sum(s_acc, axis=0, keepdims=True)

    logpt = (xt - m) - jnp.log(s)
    pt = jnp.exp(logpt)
    loss = -ALPHA * (1.0 - pt) * (1.0 - pt) * logpt
    bsum = jnp.sum(loss) * (1.0 / N_ROWS)

    @pl.when(i == 0)
    def _():
        out_ref[0, 0] = 0.0

    out_ref[0, 0] += bsum


def kernel(inputs, targets):
    xt_view = inputs.T
    t2d = targets.astype(jnp.int32).reshape(1, N_ROWS)
    out = pl.pallas_call(
        _focal_body,
        grid=(N_ROWS // BLK,),
        in_specs=[
            pl.BlockSpec((N_CLS, BLK), lambda i: (0, i)),
            pl.BlockSpec((1, BLK), lambda i: (0, i)),
        ],
        out_specs=pl.BlockSpec(
            (1, 1), lambda i: (0, 0), memory_space=pltpu.SMEM
        ),
        out_shape=jax.ShapeDtypeStruct((1, 1), jnp.float32),
        compiler_params=pltpu.CompilerParams(
            dimension_semantics=("arbitrary",),
        ),
    )(xt_view, t2d)
    return out[0, 0]


# speedup vs baseline: 1.3180x; 1.0243x over previous
import jax
import jax.numpy as jnp
from jax import lax
from jax.experimental import pallas as pl
from jax.experimental.pallas import tpu as pltpu

ALPHA = 1.0
GAMMA = 2.0
N_ROWS = 16384
N_CLS = 1000
BLK = 2048
CH = 8


def _focal_body(x_ref, t_ref, out_ref):
    i = pl.program_id(0)
    t = t_ref[...]
    sub_iota = lax.broadcasted_iota(jnp.int32, (CH, BLK), 0)
    m_acc = jnp.full((CH, BLK), -jnp.inf, jnp.float32)
    g_acc = jnp.zeros((CH, BLK), jnp.float32)
    for c in range(0, N_CLS, CH):
        xc = x_ref[c:c + CH, :]
        m_acc = jnp.maximum(m_acc, xc)
        g_acc = g_acc + jnp.where(sub_iota + c == t, xc, 0.0)
    m = jnp.max(m_acc, axis=0, keepdims=True)
    xt = jnp.sum(g_acc, axis=0, keepdims=True)

    s_acc = jnp.zeros((CH, BLK), jnp.float32)
    for c in range(0, N_CLS, CH):
        xc = x_ref[c:c + CH, :]
        s_acc = s_acc + jnp.exp(xc - m)
    s = jnp.sum(s_acc, axis=0, keepdims=True)

    logpt = (xt - m) - jnp.log(s)
    pt = jnp.exp(logpt)
    loss = -ALPHA * (1.0 - pt) * (1.0 - pt) * logpt
    bsum = jnp.sum(loss) * (1.0 / N_ROWS)

    @pl.when(i == 0)
    def _():
        out_ref[0, 0] = 0.0

    out_ref[0, 0] += bsum


def kernel(inputs, targets):
    xt_view = inputs.T
    t2d = targets.astype(jnp.int32).reshape(1, N_ROWS)
    out = pl.pallas_call(
        _focal_body,
        grid=(N_ROWS // BLK,),
        in_specs=[
            pl.BlockSpec((N_CLS, BLK), lambda i: (0, i)),
            pl.BlockSpec((1, BLK), lambda i: (0, i)),
        ],
        out_specs=pl.BlockSpec(
            (1, 1), lambda i: (0, 0), memory_space=pltpu.SMEM
        ),
        out_shape=jax.ShapeDtypeStruct((1, 1), jnp.float32),
        compiler_params=pltpu.CompilerParams(
            dimension_semantics=("arbitrary",),
        ),
    )(xt_view, t2d)
    return out[0, 0]
